# baseline (device time: 69873 ns/iter reference)
import jax
import jax.numpy as jnp
from jax import lax
from jax.experimental import pallas as pl
from jax.experimental.pallas import tpu as pltpu

M = 1024
N = 1024
BLK = 256


def kernel(dy, W):
    def body(dy_ref, w_ref, out_ref, part_ref, yrecv_ref, send_sems, recv_sems):
        mx = lax.axis_index("x")
        my = lax.axis_index("y")
        mz = lax.axis_index("z")
        b = 2 * mz + mx

        barrier = pltpu.get_barrier_semaphore()
        for nbr in ((1 - mx, my, mz), (mx, 1 - my, mz), (mx, my, 1 - mz)):
            pl.semaphore_signal(
                barrier, inc=1, device_id=nbr,
                device_id_type=pl.DeviceIdType.MESH,
            )
        pl.semaphore_wait(barrier, 3)

        dy_blk = dy_ref[pl.ds(b * BLK, BLK), :]
        part_ref[...] = lax.dot_general(
            dy_blk, w_ref[...],
            dimension_numbers=(((1,), (1,)), ((), ())),
            preferred_element_type=jnp.float32,
        )

        rdma_y = pltpu.make_async_remote_copy(
            src_ref=part_ref,
            dst_ref=yrecv_ref,
            send_sem=send_sems.at[0],
            recv_sem=recv_sems.at[0],
            device_id=(mx, 1 - my, mz),
            device_id_type=pl.DeviceIdType.MESH,
        )
        rdma_y.start()
        rdma_y.wait()
        out_ref[pl.ds(b * BLK, BLK), :] = part_ref[...] + yrecv_ref[...]

        rdma_x = pltpu.make_async_remote_copy(
            src_ref=out_ref.at[pl.ds(b * BLK, BLK), :],
            dst_ref=out_ref.at[pl.ds(b * BLK, BLK), :],
            send_sem=send_sems.at[1],
            recv_sem=recv_sems.at[1],
            device_id=(1 - mx, my, mz),
            device_id_type=pl.DeviceIdType.MESH,
        )
        rdma_x.start()
        rdma_x.wait()

        rdma_z = pltpu.make_async_remote_copy(
            src_ref=out_ref.at[pl.ds(mz * 2 * BLK, 2 * BLK), :],
            dst_ref=out_ref.at[pl.ds(mz * 2 * BLK, 2 * BLK), :],
            send_sem=send_sems.at[2],
            recv_sem=recv_sems.at[2],
            device_id=(mx, my, 1 - mz),
            device_id_type=pl.DeviceIdType.MESH,
        )
        rdma_z.start()
        rdma_z.wait()

    return pl.pallas_call(
        body,
        out_shape=jax.ShapeDtypeStruct((M, N), jnp.float32),
        in_specs=[
            pl.BlockSpec(memory_space=pltpu.VMEM),
            pl.BlockSpec(memory_space=pltpu.VMEM),
        ],
        out_specs=pl.BlockSpec(memory_space=pltpu.VMEM),
        scratch_shapes=[
            pltpu.VMEM((BLK, N), jnp.float32),
            pltpu.VMEM((BLK, N), jnp.float32),
            pltpu.SemaphoreType.DMA((3,)),
            pltpu.SemaphoreType.DMA((3,)),
        ],
        compiler_params=pltpu.CompilerParams(collective_id=0),
    )(dy, W)


# device time: 44766 ns/iter; 1.5608x vs baseline; 1.5608x over previous
import jax
import jax.numpy as jnp
from jax import lax
from jax.experimental import pallas as pl
from jax.experimental.pallas import tpu as pltpu

M = 1024
N = 1024
K = 4096
BLK = 256
NC = 4
CC = N // NC


def kernel(dy, W):
    def body(dy_ref, w_ref, out_ref,
             dyb_ref, wbuf_ref, part_ref, yrecv_ref,
             ldma_sems,
             y_send, y_recv, x_send, x_recv,
             za_send, za_recv, zb_send, zb_recv):
        mx = lax.axis_index("x")
        my = lax.axis_index("y")
        mz = lax.axis_index("z")
        b = 2 * mz + mx
        bx = 2 * mz + (1 - mx)
        row0 = b * BLK
        rowx = bx * BLK

        dy_cp = pltpu.make_async_copy(
            dy_ref.at[pl.ds(row0, BLK), :], dyb_ref, ldma_sems.at[2])
        dy_cp.start()
        w_cps = [
            pltpu.make_async_copy(
                w_ref.at[pl.ds(c * CC, CC), :], wbuf_ref.at[c % 2],
                ldma_sems.at[c % 2])
            for c in range(NC)
        ]
        w_cps[0].start()

        barrier = pltpu.get_barrier_semaphore()
        for nbr in ((1 - mx, my, mz), (mx, 1 - my, mz), (mx, my, 1 - mz)):
            pl.semaphore_signal(
                barrier, inc=1, device_id=nbr,
                device_id_type=pl.DeviceIdType.MESH,
            )
        pl.semaphore_wait(barrier, 3)

        def rdma(src, dst, ssem, rsem, dev):
            return pltpu.make_async_remote_copy(
                src_ref=src, dst_ref=dst, send_sem=ssem, recv_sem=rsem,
                device_id=dev, device_id_type=pl.DeviceIdType.MESH)

        y_nbr = (mx, 1 - my, mz)
        x_nbr = (1 - mx, my, mz)
        z_nbr = (mx, my, 1 - mz)

        y_rdmas, x_rdmas, za_rdmas, zb_rdmas = [], [], [], []
        for c in range(NC):
            cols = pl.ds(c * CC, CC)
            y_rdmas.append(rdma(part_ref.at[c], yrecv_ref.at[c],
                                y_send.at[c], y_recv.at[c], y_nbr))
            x_rdmas.append(rdma(out_ref.at[pl.ds(row0, BLK), cols],
                                out_ref.at[pl.ds(row0, BLK), cols],
                                x_send.at[c], x_recv.at[c], x_nbr))
            za_rdmas.append(rdma(out_ref.at[pl.ds(row0, BLK), cols],
                                 out_ref.at[pl.ds(row0, BLK), cols],
                                 za_send.at[c], za_recv.at[c], z_nbr))
            zb_rdmas.append(rdma(out_ref.at[pl.ds(rowx, BLK), cols],
                                 out_ref.at[pl.ds(rowx, BLK), cols],
                                 zb_send.at[c], zb_recv.at[c], z_nbr))

        dy_cp.wait()
        for c in range(NC):
            w_cps[c].wait()
            if c + 1 < NC:
                w_cps[c + 1].start()
            part_ref[c] = lax.dot_general(
                dyb_ref[...], wbuf_ref[c % 2],
                dimension_numbers=(((1,), (1,)), ((), ())),
                preferred_element_type=jnp.float32,
            )
            y_rdmas[c].start()

        for c in range(NC):
            y_rdmas[c].wait()
            out_ref[pl.ds(row0, BLK), pl.ds(c * CC, CC)] = (
                part_ref[c] + yrecv_ref[c])
            x_rdmas[c].start()
            za_rdmas[c].start()

        for c in range(NC):
            x_rdmas[c].wait()
            zb_rdmas[c].start()

        for c in range(NC):
            za_rdmas[c].wait()
            zb_rdmas[c].wait()

    return pl.pallas_call(
        body,
        out_shape=jax.ShapeDtypeStruct((M, N), jnp.float32),
        in_specs=[
            pl.BlockSpec(memory_space=pl.ANY),
            pl.BlockSpec(memory_space=pl.ANY),
        ],
        out_specs=pl.BlockSpec(memory_space=pltpu.VMEM),
        scratch_shapes=[
            pltpu.VMEM((BLK, K), jnp.float32),
            pltpu.VMEM((2, CC, K), jnp.float32),
            pltpu.VMEM((NC, BLK, CC), jnp.float32),
            pltpu.VMEM((NC, BLK, CC), jnp.float32),
            pltpu.SemaphoreType.DMA((3,)),
            pltpu.SemaphoreType.DMA((NC,)),
            pltpu.SemaphoreType.DMA((NC,)),
            pltpu.SemaphoreType.DMA((NC,)),
            pltpu.SemaphoreType.DMA((NC,)),
            pltpu.SemaphoreType.DMA((NC,)),
            pltpu.SemaphoreType.DMA((NC,)),
            pltpu.SemaphoreType.DMA((NC,)),
            pltpu.SemaphoreType.DMA((NC,)),
        ],
        compiler_params=pltpu.CompilerParams(collective_id=0),
    )(dy, W)


# device time: 39953 ns/iter; 1.7489x vs baseline; 1.1205x over previous
import jax
import jax.numpy as jnp
from jax import lax
from jax.experimental import pallas as pl
from jax.experimental.pallas import tpu as pltpu

M = 1024
N = 1024
K = 4096
BLK = 256
NC = 4
CC = N // NC


def kernel(dy, W):
    def body(dy_ref, w_ref, out_ref,
             dyb_ref, wbuf_ref, part_ref, yrecv_ref,
             ldma_sems,
             y_send, y_recv, x_send, x_recv,
             za_send, za_recv, zb_send, zb_recv, xb_send, xb_recv):
        mx = lax.axis_index("x")
        my = lax.axis_index("y")
        mz = lax.axis_index("z")
        b = 2 * mz + mx
        bx = 2 * mz + (1 - mx)
        bz = 2 * (1 - mz) + mx
        row0 = b * BLK
        rowx = bx * BLK
        rowz = bz * BLK
        HB = BLK // 2

        dy_cp = pltpu.make_async_copy(
            dy_ref.at[pl.ds(row0, BLK), :], dyb_ref, ldma_sems.at[2])
        dy_cp.start()
        w_cps = [
            pltpu.make_async_copy(
                w_ref.at[pl.ds(c * CC, CC), :], wbuf_ref.at[c % 2],
                ldma_sems.at[c % 2])
            for c in range(NC)
        ]
        w_cps[0].start()

        barrier = pltpu.get_barrier_semaphore()
        for nbr in ((1 - mx, my, mz), (mx, 1 - my, mz), (mx, my, 1 - mz)):
            pl.semaphore_signal(
                barrier, inc=1, device_id=nbr,
                device_id_type=pl.DeviceIdType.MESH,
            )
        pl.semaphore_wait(barrier, 3)

        def rdma(src, dst, ssem, rsem, dev):
            return pltpu.make_async_remote_copy(
                src_ref=src, dst_ref=dst, send_sem=ssem, recv_sem=rsem,
                device_id=dev, device_id_type=pl.DeviceIdType.MESH)

        y_nbr = (mx, 1 - my, mz)
        x_nbr = (1 - mx, my, mz)
        z_nbr = (mx, my, 1 - mz)

        y_rdmas, x_rdmas, za_rdmas, zb_rdmas, xb_rdmas = [], [], [], [], []
        for c in range(NC):
            cols = pl.ds(c * CC, CC)
            y_rdmas.append(rdma(part_ref.at[c], yrecv_ref.at[c],
                                y_send.at[c], y_recv.at[c], y_nbr))
            x_rdmas.append(rdma(out_ref.at[pl.ds(row0, BLK), cols],
                                out_ref.at[pl.ds(row0, BLK), cols],
                                x_send.at[c], x_recv.at[c], x_nbr))
            za_rdmas.append(rdma(out_ref.at[pl.ds(row0, BLK), cols],
                                 out_ref.at[pl.ds(row0, BLK), cols],
                                 za_send.at[c], za_recv.at[c], z_nbr))
            zb_rdmas.append(rdma(out_ref.at[pl.ds(rowx, HB), cols],
                                 out_ref.at[pl.ds(rowx, HB), cols],
                                 zb_send.at[c], zb_recv.at[c], z_nbr))
            xb_rdmas.append(rdma(out_ref.at[pl.ds(rowz + HB, HB), cols],
                                 out_ref.at[pl.ds(rowz + HB, HB), cols],
                                 xb_send.at[c], xb_recv.at[c], x_nbr))

        dy_cp.wait()
        for c in range(NC):
            w_cps[c].wait()
            if c + 1 < NC:
                w_cps[c + 1].start()
            part_ref[c] = lax.dot_general(
                dyb_ref[...], wbuf_ref[c % 2],
                dimension_numbers=(((1,), (1,)), ((), ())),
                preferred_element_type=jnp.float32,
            )
            y_rdmas[c].start()

        for c in range(NC):
            y_rdmas[c].wait()
            out_ref[pl.ds(row0, BLK), pl.ds(c * CC, CC)] = (
                part_ref[c] + yrecv_ref[c])
            x_rdmas[c].start()
            za_rdmas[c].start()

        for c in range(NC):
            x_rdmas[c].wait()
            zb_rdmas[c].start()

        for c in range(NC):
            za_rdmas[c].wait()
            xb_rdmas[c].start()

        for c in range(NC):
            zb_rdmas[c].wait()
            xb_rdmas[c].wait()

    return pl.pallas_call(
        body,
        out_shape=jax.ShapeDtypeStruct((M, N), jnp.float32),
        in_specs=[
            pl.BlockSpec(memory_space=pl.ANY),
            pl.BlockSpec(memory_space=pl.ANY),
        ],
        out_specs=pl.BlockSpec(memory_space=pltpu.VMEM),
        scratch_shapes=[
            pltpu.VMEM((BLK, K), jnp.float32),
            pltpu.VMEM((2, CC, K), jnp.float32),
            pltpu.VMEM((NC, BLK, CC), jnp.float32),
            pltpu.VMEM((NC, BLK, CC), jnp.float32),
            pltpu.SemaphoreType.DMA((3,)),
            pltpu.SemaphoreType.DMA((NC,)),
            pltpu.SemaphoreType.DMA((NC,)),
            pltpu.SemaphoreType.DMA((NC,)),
            pltpu.SemaphoreType.DMA((NC,)),
            pltpu.SemaphoreType.DMA((NC,)),
            pltpu.SemaphoreType.DMA((NC,)),
            pltpu.SemaphoreType.DMA((NC,)),
            pltpu.SemaphoreType.DMA((NC,)),
            pltpu.SemaphoreType.DMA((NC,)),
            pltpu.SemaphoreType.DMA((NC,)),
        ],
        compiler_params=pltpu.CompilerParams(collective_id=0),
    )(dy, W)
